# baseline (speedup 1.0000x reference)
import jax
import jax.numpy as jnp
from jax.experimental import pallas as pl
from jax.experimental.pallas import tpu as pltpu

_B = 16
_R = 576
_K = 64
_C = 1024


_B_BLK = 16


def _assign_kernel(x_ref, c_ref, out_ref):
    ct = c_ref[...]
    nc = jnp.sum(ct * ct, axis=0, keepdims=True)
    nc_col = jnp.swapaxes(nc, 0, 1)
    ct2 = ct + ct
    _NB = 4
    for p in range(_B_BLK // _NB):
        xcat = jnp.concatenate(
            [x_ref[_NB * p + i] for i in range(_NB)], axis=1)
        nx = jnp.sum(xcat * xcat, axis=0)[None, :]
        dot2 = jax.lax.dot_general(
            ct2, xcat, (((0,), (0,)), ((), ())),
            preferred_element_type=jnp.float32,
        )
        d = (nx + nc_col) - dot2
        idx = jnp.argmin(d, axis=0)
        for i in range(_NB):
            out_ref[_NB * p + i, 0, :] = idx[i * _R:(i + 1) * _R]


def kernel(x, centroids):
    xt = jnp.swapaxes(x, 1, 2)
    ct = centroids.T
    out = pl.pallas_call(
        _assign_kernel,
        grid=(_B // _B_BLK,),
        in_specs=[
            pl.BlockSpec((_B_BLK, _K, _R), lambda i: (i, 0, 0)),
            pl.BlockSpec((_K, _C), lambda i: (0, 0)),
        ],
        out_specs=pl.BlockSpec((_B_BLK, 1, _R), lambda i: (i, 0, 0)),
        out_shape=jax.ShapeDtypeStruct((_B, 1, _R), jnp.int32),
    )(xt, ct)
    return out.reshape(_B, _R)


# speedup vs baseline: 1.0040x; 1.0040x over previous
import jax
import jax.numpy as jnp
from jax.experimental import pallas as pl
from jax.experimental.pallas import tpu as pltpu

_B = 16
_R = 576
_K = 64
_C = 1024


_B_BLK = 16


def _assign_kernel(x_ref, c_ref, out_ref):
    ct = c_ref[...]
    nc = jnp.sum(ct * ct, axis=0, keepdims=True)
    nc_col = jnp.swapaxes(nc, 0, 1)
    ct2 = ct + ct
    for p in range(_B_BLK // 2):
        xcat = jnp.concatenate([x_ref[2 * p], x_ref[2 * p + 1]], axis=1)
        nx = jnp.sum(xcat * xcat, axis=0)[None, :]
        dot2 = jax.lax.dot_general(
            ct2, xcat, (((0,), (0,)), ((), ())),
            preferred_element_type=jnp.float32,
        )
        d = (nx + nc_col) - dot2
        idx = jnp.argmin(d, axis=0)
        out_ref[2 * p, 0, :] = idx[:_R]
        out_ref[2 * p + 1, 0, :] = idx[_R:]


def kernel(x, centroids):
    xt = jnp.swapaxes(x, 1, 2)
    ct = centroids.T
    out = pl.pallas_call(
        _assign_kernel,
        grid=(_B // _B_BLK,),
        in_specs=[
            pl.BlockSpec((_B_BLK, _K, _R), lambda i: (i, 0, 0)),
            pl.BlockSpec((_K, _C), lambda i: (0, 0)),
        ],
        out_specs=pl.BlockSpec((_B_BLK, 1, _R), lambda i: (i, 0, 0)),
        out_shape=jax.ShapeDtypeStruct((_B, 1, _R), jnp.int32),
    )(xt, ct)
    return out.reshape(_B, _R)
